# baseline (speedup 1.0000x reference)
import jax
import jax.numpy as jnp
from jax import lax
from jax.experimental import pallas as pl
from jax.experimental.pallas import tpu as pltpu
from jax.experimental.pallas import tpu_sc as plsc

B = 16384
F = 26
V = 100000
D = 16

NC = 2
NS = 16
NW = NC * NS

BF = B * F
PER_W = BF // NW
CHUNK = 1664
NCHUNK = PER_W // CHUNK


def _sc_body(idx_hbm, tab_hbm, out_hbm,
             idx0_v, idx1_v, rows0_v, rows1_v, i0, i1, g0, g1, w0, w1):
    wid = lax.axis_index("s") * NC + lax.axis_index("c")
    base = wid * PER_W
    pltpu.sync_copy(idx_hbm.at[pl.ds(base, CHUNK)], idx0_v)
    pltpu.sync_copy(rows0_v, out_hbm.at[pl.ds(base, CHUNK)])


@jax.jit
def _embed(sparse_ids, tables):
    flat_idx = (sparse_ids
                + jnp.arange(F, dtype=jnp.int32)[None, :] * V).reshape(BF)
    flat_tab = jnp.transpose(
        jnp.transpose(tables, (0, 2, 1)).reshape(F, 2, 8, V),
        (0, 1, 3, 2)).reshape(F * 2 * V, 8)
    mesh = plsc.VectorSubcoreMesh(core_axis_name="c", subcore_axis_name="s")
    out = pl.kernel(
        _sc_body,
        out_type=jax.ShapeDtypeStruct((BF, D), jnp.float32),
        mesh=mesh,
        scratch_types=[
            pltpu.VMEM((CHUNK,), jnp.int32),
            pltpu.VMEM((CHUNK,), jnp.int32),
            pltpu.VMEM((CHUNK, D), jnp.float32),
            pltpu.VMEM((CHUNK, D), jnp.float32),
            pltpu.SemaphoreType.DMA,
            pltpu.SemaphoreType.DMA,
            pltpu.SemaphoreType.DMA,
            pltpu.SemaphoreType.DMA,
            pltpu.SemaphoreType.DMA,
            pltpu.SemaphoreType.DMA,
        ],
        compiler_params=pltpu.CompilerParams(use_tc_tiling_on_sc=False),
    )(flat_idx, flat_tab)
    return out.reshape(B, F * D)


def kernel(sparse_ids, domain_indicator, tables):
    del domain_indicator
    return _embed(sparse_ids, tables)


# speedup vs baseline: 2.5394x; 2.5394x over previous
import jax
import jax.numpy as jnp
from jax import lax
from jax.experimental import pallas as pl
from jax.experimental.pallas import tpu as pltpu
from jax.experimental.pallas import tpu_sc as plsc

B = 16384
F = 26
V = 100000
D = 16

NC = 2
NS = 16
NW = NC * NS

NE = B * F * D
PER_W = NE // NW
CH = 16384
NCH = PER_W // CH


def _sc_body(idx_hbm, tab_hbm, out_hbm,
             idx0_v, idx1_v, buf0_v, buf1_v, i0, i1, g0, g1, w0, w1):
    wid = lax.axis_index("s") * NC + lax.axis_index("c")
    base = wid * PER_W

    idx_bufs = [idx0_v, idx1_v]
    word_bufs = [buf0_v, buf1_v]
    isems = [i0, i1]
    gsems = [g0, g1]
    wsems = [w0, w1]

    def stage_idx(c):
        return pltpu.async_copy(
            idx_hbm.at[pl.ds(base + c * CH, CH)], idx_bufs[c & 1],
            isems[c & 1])

    stages = [None] * NCH
    gathers = [None] * NCH
    writes = [None] * NCH

    stages[0] = stage_idx(0)
    stages[1] = stage_idx(1)
    stages[0].wait()
    gathers[0] = pltpu.async_copy(tab_hbm.at[idx_bufs[0]], word_bufs[0], gsems[0])

    for c in range(NCH):
        nb = c & 1
        gathers[c].wait()
        writes[c] = pltpu.async_copy(
            word_bufs[nb], out_hbm.at[pl.ds(base + c * CH, CH)], wsems[nb])
        if c + 1 < NCH:
            stages[c + 1].wait()
            if c >= 1:
                writes[c - 1].wait()
            gathers[c + 1] = pltpu.async_copy(
                tab_hbm.at[idx_bufs[(c + 1) & 1]], word_bufs[(c + 1) & 1],
                gsems[(c + 1) & 1])
            if c + 2 < NCH:
                stages[c + 2] = stage_idx(c + 2)

    writes[NCH - 2].wait()
    writes[NCH - 1].wait()


@jax.jit
def _embed(sparse_ids, tables):
    plane = (jnp.arange(F, dtype=jnp.int32)[:, None] * D
             + jnp.arange(D, dtype=jnp.int32)[None, :]) * V
    idx_elem = (sparse_ids[:, :, None] + plane[None]).reshape(NE)
    tab_lin = jnp.transpose(tables, (0, 2, 1)).reshape(F * D * V)
    mesh = plsc.VectorSubcoreMesh(core_axis_name="c", subcore_axis_name="s")
    out = pl.kernel(
        _sc_body,
        out_type=jax.ShapeDtypeStruct((NE,), jnp.float32),
        mesh=mesh,
        scratch_types=[
            pltpu.VMEM((CH,), jnp.int32),
            pltpu.VMEM((CH,), jnp.int32),
            pltpu.VMEM((CH,), jnp.float32),
            pltpu.VMEM((CH,), jnp.float32),
            pltpu.SemaphoreType.DMA,
            pltpu.SemaphoreType.DMA,
            pltpu.SemaphoreType.DMA,
            pltpu.SemaphoreType.DMA,
            pltpu.SemaphoreType.DMA,
            pltpu.SemaphoreType.DMA,
        ],
        compiler_params=pltpu.CompilerParams(use_tc_tiling_on_sc=False),
    )(idx_elem, tab_lin)
    return out.reshape(B, F * D)


def kernel(sparse_ids, domain_indicator, tables):
    del domain_indicator
    return _embed(sparse_ids, tables)


# speedup vs baseline: 2.5444x; 1.0019x over previous
import jax
import jax.numpy as jnp
from jax import lax
from jax.experimental import pallas as pl
from jax.experimental.pallas import tpu as pltpu
from jax.experimental.pallas import tpu_sc as plsc

B = 16384
F = 26
V = 100000
D = 16

NC = 2
NS = 16
NW = NC * NS

NE = B * F * D
PER_W = NE // NW
CH = 26624
NCH = PER_W // CH


def _sc_body(idx_hbm, tab_hbm, out_hbm,
             idx0_v, idx1_v, buf0_v, buf1_v, i0, i1, g0, g1, w0, w1):
    wid = lax.axis_index("s") * NC + lax.axis_index("c")
    base = wid * PER_W

    idx_bufs = [idx0_v, idx1_v]
    word_bufs = [buf0_v, buf1_v]
    isems = [i0, i1]
    gsems = [g0, g1]
    wsems = [w0, w1]

    def stage_idx(c):
        return pltpu.async_copy(
            idx_hbm.at[pl.ds(base + c * CH, CH)], idx_bufs[c & 1],
            isems[c & 1])

    stages = [None] * NCH
    gathers = [None] * NCH
    writes = [None] * NCH

    stages[0] = stage_idx(0)
    stages[1] = stage_idx(1)
    stages[0].wait()
    gathers[0] = pltpu.async_copy(tab_hbm.at[idx_bufs[0]], word_bufs[0], gsems[0])

    for c in range(NCH):
        nb = c & 1
        gathers[c].wait()
        writes[c] = pltpu.async_copy(
            word_bufs[nb], out_hbm.at[pl.ds(base + c * CH, CH)], wsems[nb])
        if c + 1 < NCH:
            stages[c + 1].wait()
            if c >= 1:
                writes[c - 1].wait()
            gathers[c + 1] = pltpu.async_copy(
                tab_hbm.at[idx_bufs[(c + 1) & 1]], word_bufs[(c + 1) & 1],
                gsems[(c + 1) & 1])
            if c + 2 < NCH:
                stages[c + 2] = stage_idx(c + 2)

    writes[NCH - 2].wait()
    writes[NCH - 1].wait()


@jax.jit
def _embed(sparse_ids, tables):
    plane = (jnp.arange(F, dtype=jnp.int32)[:, None] * D
             + jnp.arange(D, dtype=jnp.int32)[None, :]) * V
    idx_elem = (sparse_ids[:, :, None] + plane[None]).reshape(NE)
    tab_lin = jnp.transpose(tables, (0, 2, 1)).reshape(F * D * V)
    mesh = plsc.VectorSubcoreMesh(core_axis_name="c", subcore_axis_name="s")
    out = pl.kernel(
        _sc_body,
        out_type=jax.ShapeDtypeStruct((NE,), jnp.float32),
        mesh=mesh,
        scratch_types=[
            pltpu.VMEM((CH,), jnp.int32),
            pltpu.VMEM((CH,), jnp.int32),
            pltpu.VMEM((CH,), jnp.float32),
            pltpu.VMEM((CH,), jnp.float32),
            pltpu.SemaphoreType.DMA,
            pltpu.SemaphoreType.DMA,
            pltpu.SemaphoreType.DMA,
            pltpu.SemaphoreType.DMA,
            pltpu.SemaphoreType.DMA,
            pltpu.SemaphoreType.DMA,
        ],
        compiler_params=pltpu.CompilerParams(use_tc_tiling_on_sc=False),
    )(idx_elem, tab_lin)
    return out.reshape(B, F * D)


def kernel(sparse_ids, domain_indicator, tables):
    del domain_indicator
    return _embed(sparse_ids, tables)


# speedup vs baseline: 2.5575x; 1.0052x over previous
import jax
import jax.numpy as jnp
from jax import lax
from jax.experimental import pallas as pl
from jax.experimental.pallas import tpu as pltpu
from jax.experimental.pallas import tpu_sc as plsc

B = 16384
F = 26
V = 100000
D = 16

NC = 2
NS = 16
NW = NC * NS

NE = B * F * D
PER_W = NE // NW
CH = 13312
NCH = PER_W // CH


def _sc_body(idx_hbm, tab_hbm, out_hbm,
             idx0_v, idx1_v, idx2_v, idx3_v, buf0_v, buf1_v, buf2_v, buf3_v,
             i0, i1, i2, i3, g0, g1, g2, g3, w0, w1, w2, w3):
    wid = lax.axis_index("s") * NC + lax.axis_index("c")
    base = wid * PER_W

    idx_bufs = [idx0_v, idx1_v, idx2_v, idx3_v]
    word_bufs = [buf0_v, buf1_v, buf2_v, buf3_v]
    isems = [i0, i1, i2, i3]
    gsems = [g0, g1, g2, g3]
    wsems = [w0, w1, w2, w3]

    def stage_idx(c):
        return pltpu.async_copy(
            idx_hbm.at[pl.ds(base + c * CH, CH)], idx_bufs[c % 4],
            isems[c % 4])

    stages = [None] * NCH
    gathers = [None] * NCH
    writes = [None] * NCH

    for k in range(4):
        stages[k] = stage_idx(k)
    for k in range(3):
        stages[k].wait()
        gathers[k] = pltpu.async_copy(
            tab_hbm.at[idx_bufs[k]], word_bufs[k], gsems[k])

    for c in range(NCH):
        gathers[c].wait()
        writes[c] = pltpu.async_copy(
            word_bufs[c % 4], out_hbm.at[pl.ds(base + c * CH, CH)],
            wsems[c % 4])
        if c + 3 < NCH:
            stages[c + 3].wait()
            if c >= 1:
                writes[c - 1].wait()
            gathers[c + 3] = pltpu.async_copy(
                tab_hbm.at[idx_bufs[(c + 3) % 4]], word_bufs[(c + 3) % 4],
                gsems[(c + 3) % 4])
            if c + 4 < NCH:
                stages[c + 4] = stage_idx(c + 4)

    for c in range(NCH - 4, NCH):
        writes[c].wait()


@jax.jit
def _embed(sparse_ids, tables):
    plane = (jnp.arange(F, dtype=jnp.int32)[:, None] * D
             + jnp.arange(D, dtype=jnp.int32)[None, :]) * V
    idx_elem = (sparse_ids[:, :, None] + plane[None]).reshape(NE)
    tab_lin = jnp.transpose(tables, (0, 2, 1)).reshape(F * D * V)
    mesh = plsc.VectorSubcoreMesh(core_axis_name="c", subcore_axis_name="s")
    out = pl.kernel(
        _sc_body,
        out_type=jax.ShapeDtypeStruct((NE,), jnp.float32),
        mesh=mesh,
        scratch_types=(
            [pltpu.VMEM((CH,), jnp.int32)] * 4
            + [pltpu.VMEM((CH,), jnp.float32)] * 4
            + [pltpu.SemaphoreType.DMA] * 12
        ),
        compiler_params=pltpu.CompilerParams(use_tc_tiling_on_sc=False),
    )(idx_elem, tab_lin)
    return out.reshape(B, F * D)


def kernel(sparse_ids, domain_indicator, tables):
    del domain_indicator
    return _embed(sparse_ids, tables)
